# Initial kernel scaffold; baseline (speedup 1.0000x reference)
import jax
import jax.numpy as jnp
from jax.experimental import pallas as pl


def kernel(idx, table):
    raise NotImplementedError("write your pallas kernel here")


# speedup vs baseline: 1.0134x; 1.0134x over previous
import functools

import jax
import jax.numpy as jnp
from jax import lax
from jax.experimental import pallas as pl
from jax.experimental.pallas import tpu as pltpu
from jax.experimental.pallas import tpu_sc as plsc

_VOCAB = 1000
_B, _T = 1024, 50
_BT = _B * _T

_info = plsc.get_sparse_core_info()
_NC, _NS = _info.num_cores, _info.num_subcores
_NW = _NC * _NS
_BPW = _BT // _NW
_CH = 64
_NCHUNK = _BPW // _CH
_STAGE = _CH * _VOCAB

_mesh = plsc.VectorSubcoreMesh(core_axis_name="c", subcore_axis_name="s")


@functools.partial(
    pl.kernel,
    mesh=_mesh,
    out_type=jax.ShapeDtypeStruct((_BT * _VOCAB,), jnp.float32),
    scratch_types=[
        pltpu.VMEM((_BPW,), jnp.int32),
        pltpu.VMEM((_STAGE,), jnp.float32),
        pltpu.SemaphoreType.DMA,
    ],
)
def _gather_rows(idx_hbm, table_hbm, out_hbm, idx_v, stage_v, sem):
    wid = lax.axis_index("s") * _NC + lax.axis_index("c")
    base = wid * _BPW
    pltpu.sync_copy(idx_hbm.at[pl.ds(base, _BPW)], idx_v)

    def chunk(c, carry):
        def fire(g, carry2):
            vec = idx_v[pl.ds(c * _CH + g * 16, 16)]
            for l in range(16):
                src = pl.multiple_of(vec[l] * _VOCAB, 8)
                dst = pl.multiple_of((g * 16 + l) * _VOCAB, 8)
                pltpu.async_copy(
                    table_hbm.at[pl.ds(src, _VOCAB)],
                    stage_v.at[pl.ds(dst, _VOCAB)],
                    sem,
                )
            return carry2

        lax.fori_loop(0, _CH // 16, fire, 0)
        pltpu.make_async_copy(
            table_hbm.at[pl.ds(0, _STAGE)],
            stage_v.at[pl.ds(0, _STAGE)],
            sem,
        ).wait()
        out_off = pl.multiple_of((base + c * _CH) * _VOCAB, 8)
        pltpu.sync_copy(stage_v, out_hbm.at[pl.ds(out_off, _STAGE)])
        return carry

    lax.fori_loop(0, _NCHUNK, chunk, 0)


def kernel(idx, table):
    flat = idx.reshape(-1).astype(jnp.int32)
    out = _gather_rows(flat, table.reshape(-1))
    return out.reshape(idx.shape + (_VOCAB,))
